# baseline (speedup 1.0000x reference)
import functools

import jax
import jax.numpy as jnp
from jax.experimental import pallas as pl
from jax.experimental.pallas import tpu as pltpu

B = 16
N = 32768
IN_CH = 256
ATT = 128
NEG_INF = -1e30

_DN_T = (((1,), (1,)), ((), ()))


def _fused_kernel(lo_ref, hi_ref, x_ref, wc_ref, bc_ref, ws_ref, bs_ref,
                  wm1_ref, bm1_ref, wm2_ref, bm2_ref,
                  out_ref, m_ref, s_ref, acc_ref, *, tb, nblocks):
    i = pl.program_id(0)

    @pl.when(i == 0)
    def _init():
        m_ref[...] = jnp.full((B, 1), NEG_INF, jnp.float32)
        s_ref[...] = jnp.zeros((B, 1), jnp.float32)
        acc_ref[...] = jnp.zeros((B, IN_CH), jnp.float32)

    xb = x_ref[...]
    r = jax.lax.dot_general(xb, wc_ref[...], _DN_T,
                            preferred_element_type=jnp.float32) + bc_ref[...]
    g = jnp.tanh(r[:, :ATT]) * (0.5 * jnp.tanh(r[:, ATT:]) + 0.5)
    score = jax.lax.dot_general(ws_ref[...], g, _DN_T,
                                preferred_element_type=jnp.float32) + bs_ref[...]

    pos = i * tb + jax.lax.broadcasted_iota(jnp.int32, (1, tb), 1)
    onehot = (pos >= lo_ref[...]) & (pos < hi_ref[...])

    m_old = m_ref[...]
    sm = jnp.max(jnp.where(onehot, score, NEG_INF), axis=1, keepdims=True)
    m_new = jnp.maximum(m_old, sm)
    scale = jnp.exp(m_old - m_new)

    m_tok = jnp.max(jnp.where(onehot, m_new, NEG_INF), axis=0, keepdims=True)
    e_tok = jnp.exp(score - m_tok)
    e = jnp.where(onehot, e_tok, 0.0)

    s_ref[...] = s_ref[...] * scale + jnp.sum(e, axis=1, keepdims=True)
    acc_ref[...] = acc_ref[...] * scale + jax.lax.dot_general(
        e, xb, (((1,), (0,)), ((), ())), preferred_element_type=jnp.float32)
    m_ref[...] = m_new

    @pl.when(i == nblocks - 1)
    def _finish():
        slice_x = acc_ref[...] / s_ref[...]
        h = jax.nn.relu(jax.lax.dot_general(
            slice_x, wm1_ref[...], _DN_T,
            preferred_element_type=jnp.float32) + bm1_ref[...])
        pred = jax.nn.relu(jnp.dot(
            h, wm2_ref[...],
            preferred_element_type=jnp.float32) + bm2_ref[...])
        out_ref[...] = pred


@functools.partial(jax.jit, static_argnames=("tb",))
def _run(x, idx, W1, b1, W2, b2, Ws, bs, Wm1, bm1, Wm2, bm2, tb=8192):
    nblocks = N // tb
    lo = jnp.concatenate([jnp.zeros((1,), idx.dtype), idx[:-1]]).reshape(B, 1)
    hi = idx.reshape(B, 1)
    Wc = jnp.concatenate([W1, 0.5 * W2], axis=0)
    bc = jnp.concatenate([b1, 0.5 * b2]).reshape(1, 2 * ATT)

    in_specs = [
        pl.BlockSpec((B, 1), lambda i: (0, 0)),
        pl.BlockSpec((B, 1), lambda i: (0, 0)),
        pl.BlockSpec((tb, IN_CH), lambda i: (i, 0)),
        pl.BlockSpec((2 * ATT, IN_CH), lambda i: (0, 0)),
        pl.BlockSpec((1, 2 * ATT), lambda i: (0, 0)),
        pl.BlockSpec((1, ATT), lambda i: (0, 0)),
        pl.BlockSpec((1, 1), lambda i: (0, 0)),
        pl.BlockSpec((IN_CH // 8, IN_CH), lambda i: (0, 0)),
        pl.BlockSpec((1, IN_CH // 8), lambda i: (0, 0)),
        pl.BlockSpec((IN_CH // 8, 1), lambda i: (0, 0)),
        pl.BlockSpec((1, 1), lambda i: (0, 0)),
    ]

    return pl.pallas_call(
        functools.partial(_fused_kernel, tb=tb, nblocks=nblocks),
        grid=(nblocks,),
        in_specs=in_specs,
        out_specs=pl.BlockSpec((B, 1), lambda i: (0, 0)),
        out_shape=jax.ShapeDtypeStruct((B, 1), jnp.float32),
        scratch_shapes=[
            pltpu.VMEM((B, 1), jnp.float32),
            pltpu.VMEM((B, 1), jnp.float32),
            pltpu.VMEM((B, IN_CH), jnp.float32),
        ],
        compiler_params=pltpu.CompilerParams(
            dimension_semantics=("arbitrary",),
        ),
    )(lo, hi, x, Wc, bc, Ws, bs.reshape(1, 1),
      Wm1, bm1.reshape(1, IN_CH // 8), Wm2.T, bm2.reshape(1, 1))


def kernel(x, idx, W1, b1, W2, b2, Ws, bs, Wm1, bm1, Wm2, bm2):
    return _run(x, idx, W1, b1, W2, b2, Ws, bs, Wm1, bm1, Wm2, bm2)


# speedup vs baseline: 1.0532x; 1.0532x over previous
import functools

import jax
import jax.numpy as jnp
from jax.experimental import pallas as pl
from jax.experimental.pallas import tpu as pltpu

B = 16
N = 32768
IN_CH = 256
ATT = 128
NEG_INF = -1e30

_DN_T = (((1,), (1,)), ((), ()))


def _fused_kernel(lo_ref, hi_ref, x_ref, wc_ref, bc_ref, ws_ref, bs_ref,
                  wm1_ref, bm1_ref, wm2_ref, bm2_ref,
                  out_ref, m_ref, s_ref, acc_ref, *, tb, nblocks):
    i = pl.program_id(0)

    @pl.when(i == 0)
    def _init():
        m_ref[...] = jnp.full((B, 1), NEG_INF, jnp.float32)
        s_ref[...] = jnp.zeros((B, 1), jnp.float32)
        acc_ref[...] = jnp.zeros((B, IN_CH), jnp.float32)

    xb = x_ref[...]
    r = jax.lax.dot_general(xb, wc_ref[...], _DN_T,
                            preferred_element_type=jnp.float32) + bc_ref[...]
    g = jnp.tanh(r[:, :ATT]) * jax.nn.sigmoid(r[:, ATT:])
    score = jax.lax.dot_general(ws_ref[...], g, _DN_T,
                                preferred_element_type=jnp.float32) + bs_ref[...]

    pos = i * tb + jax.lax.broadcasted_iota(jnp.int32, (1, tb), 1)
    onehot = (pos >= lo_ref[...]) & (pos < hi_ref[...])

    m_old = m_ref[...]
    sm = jnp.max(jnp.where(onehot, score, NEG_INF), axis=1, keepdims=True)
    m_new = jnp.maximum(m_old, sm)
    scale = jnp.exp(m_old - m_new)

    m_tok = jnp.max(jnp.where(onehot, m_new, NEG_INF), axis=0, keepdims=True)
    e_tok = jnp.exp(score - m_tok)
    e = jnp.where(onehot, e_tok, 0.0)

    s_ref[...] = s_ref[...] * scale + jnp.sum(e, axis=1, keepdims=True)
    acc_ref[...] = acc_ref[...] * scale + jax.lax.dot_general(
        e, xb, (((1,), (0,)), ((), ())), preferred_element_type=jnp.float32)
    m_ref[...] = m_new

    @pl.when(i == nblocks - 1)
    def _finish():
        slice_x = acc_ref[...] / s_ref[...]
        h = jax.nn.relu(jax.lax.dot_general(
            slice_x, wm1_ref[...], _DN_T,
            preferred_element_type=jnp.float32) + bm1_ref[...])
        pred = jax.nn.relu(jnp.dot(
            h, wm2_ref[...],
            preferred_element_type=jnp.float32) + bm2_ref[...])
        out_ref[...] = pred


@functools.partial(jax.jit, static_argnames=("tb",))
def _run(x, idx, W1, b1, W2, b2, Ws, bs, Wm1, bm1, Wm2, bm2, tb=8192):
    nblocks = N // tb
    lo = jnp.concatenate([jnp.zeros((1,), idx.dtype), idx[:-1]]).reshape(B, 1)
    hi = idx.reshape(B, 1)
    Wc = jnp.concatenate([W1, W2], axis=0)
    bc = jnp.concatenate([b1, b2]).reshape(1, 2 * ATT)

    in_specs = [
        pl.BlockSpec((B, 1), lambda i: (0, 0)),
        pl.BlockSpec((B, 1), lambda i: (0, 0)),
        pl.BlockSpec((tb, IN_CH), lambda i: (i, 0)),
        pl.BlockSpec((2 * ATT, IN_CH), lambda i: (0, 0)),
        pl.BlockSpec((1, 2 * ATT), lambda i: (0, 0)),
        pl.BlockSpec((1, ATT), lambda i: (0, 0)),
        pl.BlockSpec((1, 1), lambda i: (0, 0)),
        pl.BlockSpec((IN_CH // 8, IN_CH), lambda i: (0, 0)),
        pl.BlockSpec((1, IN_CH // 8), lambda i: (0, 0)),
        pl.BlockSpec((IN_CH // 8, 1), lambda i: (0, 0)),
        pl.BlockSpec((1, 1), lambda i: (0, 0)),
    ]

    return pl.pallas_call(
        functools.partial(_fused_kernel, tb=tb, nblocks=nblocks),
        grid=(nblocks,),
        in_specs=in_specs,
        out_specs=pl.BlockSpec((B, 1), lambda i: (0, 0)),
        out_shape=jax.ShapeDtypeStruct((B, 1), jnp.float32),
        scratch_shapes=[
            pltpu.VMEM((B, 1), jnp.float32),
            pltpu.VMEM((B, 1), jnp.float32),
            pltpu.VMEM((B, IN_CH), jnp.float32),
        ],
        compiler_params=pltpu.CompilerParams(
            dimension_semantics=("arbitrary",),
        ),
    )(lo, hi, x, Wc, bc, Ws, bs.reshape(1, 1),
      Wm1, bm1.reshape(1, IN_CH // 8), Wm2.T, bm2.reshape(1, 1))


def kernel(x, idx, W1, b1, W2, b2, Ws, bs, Wm1, bm1, Wm2, bm2):
    return _run(x, idx, W1, b1, W2, b2, Ws, bs, Wm1, bm1, Wm2, bm2)


# speedup vs baseline: 1.2238x; 1.1619x over previous
import functools

import jax
import jax.numpy as jnp
from jax.experimental import pallas as pl
from jax.experimental.pallas import tpu as pltpu

B = 16
N = 32768
IN_CH = 256
ATT = 128
NEG_INF = -1e30

_DN_T = (((1,), (1,)), ((), ()))


def _to_col(row):
    ii = jax.lax.broadcasted_iota(jnp.int32, (B, B), 0)
    jj = jax.lax.broadcasted_iota(jnp.int32, (B, B), 1)
    return jnp.sum(jnp.where(ii == jj, jnp.broadcast_to(row, (B, B)), 0),
                   axis=1, keepdims=True)


def _fused_kernel(idx_ref, x_ref, w1_ref, b1_ref, w2_ref, b2_ref, ws_ref,
                  bs_ref, wm1_ref, bm1_ref, wm2_ref, bm2_ref,
                  out_ref, m_ref, s_ref, acc_ref, *, tb, nblocks):
    i = pl.program_id(0)

    @pl.when(i == 0)
    def _init():
        m_ref[...] = jnp.full((B, 1), NEG_INF, jnp.float32)
        s_ref[...] = jnp.zeros((B, 1), jnp.float32)
        acc_ref[...] = jnp.zeros((B, IN_CH), jnp.float32)

    xb = x_ref[...]
    rv = jax.lax.dot_general(xb, w1_ref[...], _DN_T,
                             preferred_element_type=jnp.float32) + b1_ref[...]
    ru = jax.lax.dot_general(xb, w2_ref[...], _DN_T,
                             preferred_element_type=jnp.float32) + b2_ref[...]
    g = jnp.tanh(rv) * jax.nn.sigmoid(ru)
    score = jax.lax.dot_general(ws_ref[...], g, _DN_T,
                                preferred_element_type=jnp.float32) + bs_ref[...]

    hi_row = idx_ref[...]
    hi = _to_col(hi_row)
    lo = jnp.concatenate([jnp.zeros((1, 1), jnp.int32), hi[:B - 1, :]], axis=0)

    pos = i * tb + jax.lax.broadcasted_iota(jnp.int32, (1, tb), 1)
    onehot = (pos >= lo) & (pos < hi)

    m_old = m_ref[...]
    sm = jnp.max(jnp.where(onehot, score, NEG_INF), axis=1, keepdims=True)
    m_new = jnp.maximum(m_old, sm)
    scale = jnp.exp(m_old - m_new)

    m_tok = jnp.max(jnp.where(onehot, m_new, NEG_INF), axis=0, keepdims=True)
    e_tok = jnp.exp(score - m_tok)
    e = jnp.where(onehot, e_tok, 0.0)

    s_ref[...] = s_ref[...] * scale + jnp.sum(e, axis=1, keepdims=True)
    acc_ref[...] = acc_ref[...] * scale + jax.lax.dot_general(
        e, xb, (((1,), (0,)), ((), ())), preferred_element_type=jnp.float32)
    m_ref[...] = m_new

    @pl.when(i == nblocks - 1)
    def _finish():
        slice_x = acc_ref[...] / s_ref[...]
        h = jax.nn.relu(jax.lax.dot_general(
            slice_x, wm1_ref[...], _DN_T,
            preferred_element_type=jnp.float32) + bm1_ref[...])
        out_ref[...] = jax.nn.relu(jax.lax.dot_general(
            wm2_ref[...], h, _DN_T,
            preferred_element_type=jnp.float32) + bm2_ref[...])


@functools.partial(jax.jit, static_argnames=("tb",))
def _run(x, idx, W1, b1, W2, b2, Ws, bs, Wm1, bm1, Wm2, bm2, tb=8192):
    nblocks = N // tb

    in_specs = [
        pl.BlockSpec((1, B), lambda i: (0, 0)),
        pl.BlockSpec((tb, IN_CH), lambda i: (i, 0)),
        pl.BlockSpec((ATT, IN_CH), lambda i: (0, 0)),
        pl.BlockSpec((1, ATT), lambda i: (0, 0)),
        pl.BlockSpec((ATT, IN_CH), lambda i: (0, 0)),
        pl.BlockSpec((1, ATT), lambda i: (0, 0)),
        pl.BlockSpec((1, ATT), lambda i: (0, 0)),
        pl.BlockSpec((1, 1), lambda i: (0, 0)),
        pl.BlockSpec((IN_CH // 8, IN_CH), lambda i: (0, 0)),
        pl.BlockSpec((1, IN_CH // 8), lambda i: (0, 0)),
        pl.BlockSpec((1, IN_CH // 8), lambda i: (0, 0)),
        pl.BlockSpec((1, 1), lambda i: (0, 0)),
    ]

    pred_row = pl.pallas_call(
        functools.partial(_fused_kernel, tb=tb, nblocks=nblocks),
        grid=(nblocks,),
        in_specs=in_specs,
        out_specs=pl.BlockSpec((1, B), lambda i: (0, 0)),
        out_shape=jax.ShapeDtypeStruct((1, B), jnp.float32),
        scratch_shapes=[
            pltpu.VMEM((B, 1), jnp.float32),
            pltpu.VMEM((B, 1), jnp.float32),
            pltpu.VMEM((B, IN_CH), jnp.float32),
        ],
        compiler_params=pltpu.CompilerParams(
            dimension_semantics=("arbitrary",),
        ),
    )(idx.reshape(1, B), x, W1, b1.reshape(1, ATT), W2, b2.reshape(1, ATT),
      Ws, bs.reshape(1, 1), Wm1, bm1.reshape(1, IN_CH // 8), Wm2,
      bm2.reshape(1, 1))
    return pred_row.reshape(B, 1)


def kernel(x, idx, W1, b1, W2, b2, Ws, bs, Wm1, bm1, Wm2, bm2):
    return _run(x, idx, W1, b1, W2, b2, Ws, bs, Wm1, bm1, Wm2, bm2)
